# Initial kernel scaffold; baseline (speedup 1.0000x reference)
import jax
import jax.numpy as jnp
from jax.experimental import pallas as pl


def kernel(x, conv1_w, conv1_b, conv2_w, conv2_b, pool_w, lin1_w, lin1_b, lin2_w, lin2_b, cls1_w, cls1_b, cls2_w, cls2_b):
    raise NotImplementedError("write your pallas kernel here")


# speedup vs baseline: 2.4535x; 2.4535x over previous
import functools

import jax
import jax.numpy as jnp
import numpy as np
from jax.experimental import pallas as pl
from jax.experimental.pallas import tpu as pltpu

IN_C, DIM, OUT_C, N_PIECES = 3, 256, 256, 20
BATCH, SET_N = 16, 128

_IC = 16
_BC = 16


def _interp_matrix() -> np.ndarray:
    n = SET_N
    ratios = np.clip(np.arange(n, dtype=np.float64) / max(n - 1, 1), 0.0, 1.0)
    idx_f = N_PIECES * ratios
    idx = np.floor(idx_f).astype(np.int64)
    frac = (idx_f - idx).astype(np.float32)
    g = np.zeros((n, N_PIECES + 1), np.float32)
    g[np.arange(n), idx] += 1.0 - frac
    g[np.arange(n), np.clip(idx + 1, 0, N_PIECES)] += frac
    return g


def _fused_kernel(xT_ref, w1t_ref, b1_ref, w2t_ref, b2_ref,
                  g_ref, poolwT_ref, scal_ref,
                  l1t_ref, l1b_ref, l2t_ref, l2b_ref,
                  c1t_ref, c1b_ref, c2t_ref, c2b_ref,
                  out_ref, sT_ref, bs_ref, wt_ref):
    n = SET_N
    bf = jnp.bfloat16
    xb = xT_ref[0].astype(bf)
    h = jnp.dot(xb, w1t_ref[...].astype(bf), preferred_element_type=jnp.float32)
    h = jnp.maximum(h + b1_ref[...], 0.0)
    sT_ref[...] = jnp.dot(h.astype(bf), w2t_ref[...].astype(bf),
                          preferred_element_type=jnp.float32) + b2_ref[...]
    wt_ref[...] = jnp.dot(g_ref[...], poolwT_ref[...],
                          preferred_element_type=jnp.float32, precision=jax.lax.Precision.HIGHEST)

    bs_ref[...] = jnp.zeros((n, DIM), jnp.float32)

    def bs_body(k, _):
        i0 = k * _BC
        blk = sT_ref[pl.ds(i0, _BC), :]
        d = jnp.abs(sT_ref[...][None, :, :] - blk[:, None, :])
        bs_ref[...] += d.sum(axis=0)
        return 0

    jax.lax.fori_loop(0, n // _BC, bs_body, 0)

    def sm_body(k, pooled):
        i0 = k * _IC
        sT = sT_ref[...]
        sc = scal_ref[pl.ds(i0, _IC), :]
        logits = sc[:, None, :] * sT[None, :, :] - bs_ref[...][None, :, :]
        m = jnp.max(logits, axis=1, keepdims=True)
        e = jnp.exp(logits - m)
        den = jnp.sum(e, axis=1, keepdims=True)
        p = (e / den).astype(jnp.bfloat16).astype(jnp.float32)
        sb = sT.astype(jnp.bfloat16).astype(jnp.float32)
        xs = jnp.sum(p * sb[None, :, :], axis=1, keepdims=True)
        wc = wt_ref[pl.ds(i0, _IC), :]
        return pooled + (wc[:, None, :] * xs).sum(axis=0)

    pooled = jax.lax.fori_loop(0, n // _IC, sm_body,
                               jnp.zeros((1, DIM), jnp.float32))

    z = jnp.dot(pooled.astype(bf), l1t_ref[...].astype(bf),
                preferred_element_type=jnp.float32)
    z = jnp.maximum(z + l1b_ref[...], 0.0)
    z = jnp.dot(z.astype(bf), l2t_ref[...].astype(bf),
                preferred_element_type=jnp.float32) + l2b_ref[...]
    z = jnp.maximum(jnp.dot(z.astype(bf), c1t_ref[...].astype(bf),
                            preferred_element_type=jnp.float32) + c1b_ref[...], 0.0)
    o = jnp.dot(z.astype(bf), c2t_ref[...].astype(bf),
                preferred_element_type=jnp.float32) + c2b_ref[...]
    out_ref[...] = o.reshape(1, 1, 10)


@functools.partial(jax.jit, static_argnames=("interpret",))
def kernel(x, conv1_w, conv1_b, conv2_w, conv2_b, pool_w,
           lin1_w, lin1_b, lin2_w, lin2_b, cls1_w, cls1_b, cls2_w, cls2_b,
           interpret=False):
    f32 = jnp.float32
    n = SET_N
    xT = x.transpose(0, 2, 1)
    g = jnp.asarray(_interp_matrix())
    scal = (n - 1 - 2.0 * jnp.arange(n, dtype=f32))[:, None] * jnp.ones((1, DIM), f32)
    row = lambda v: v.reshape(1, -1)

    full = lambda a: pl.BlockSpec(a.shape, lambda b: (0,) * a.ndim)
    args = (xT, conv1_w.T, row(conv1_b), conv2_w.T, row(conv2_b),
            g, pool_w.T, scal,
            lin1_w.T, row(lin1_b), lin2_w.T, row(lin2_b),
            cls1_w.T, row(cls1_b), cls2_w.T, row(cls2_b))
    in_specs = [pl.BlockSpec((1, n, IN_C), lambda b: (b, 0, 0))]
    in_specs += [full(a) for a in args[1:]]

    out = pl.pallas_call(
        _fused_kernel,
        grid=(BATCH,),
        in_specs=in_specs,
        out_specs=pl.BlockSpec((1, 1, 10), lambda b: (b, 0, 0)),
        out_shape=jax.ShapeDtypeStruct((BATCH, 1, 10), f32),
        scratch_shapes=[
            pltpu.VMEM((n, DIM), f32),
            pltpu.VMEM((n, DIM), f32),
            pltpu.VMEM((n, DIM), f32),
        ],
        compiler_params=pltpu.CompilerParams(
            dimension_semantics=("parallel",),
        ),
        name="fsencoder_fused",
        interpret=interpret,
    )(*args)
    return out.reshape(BATCH, 10)
